# baseline (speedup 1.0000x reference)
import math

import jax
import jax.numpy as jnp
from jax import lax
from jax.experimental import pallas as pl
from jax.experimental.pallas import tpu as pltpu
from jax.experimental.pallas import tpu_sc as plsc

VOCAB = 100000
HIDDEN = 128
B = 1024
L = 200
NC = 2
NS = 16
NW = NC * NS
B_PER_W = B // NW
SCALE = 1.0 / math.sqrt(HIDDEN)
NVH = HIDDEN // 16
XB = 6
NSL = 4


def _emb_body(x_hbm, table_hbm, pos_hbm, out_hbm,
              pos_v, buf0, buf1, buf2, buf3, xq0, xq1,
              gsem0, gsem1, gsem2, gsem3, osem0, osem1, osem2, osem3, xsem):
    wid = lax.axis_index("s") * NC + lax.axis_index("c")
    base = wid * B_PER_W

    bufs = (buf0, buf1, buf2, buf3)
    gsems = (gsem0, gsem1, gsem2, gsem3)
    osems = (osem0, osem1, osem2, osem3)
    xqs = (xq0, xq1)

    out_cp = [None] * NSL
    gather_cp = [None] * NSL
    xfill_cp = [None, None]

    def start_xfill(s):
        nb = min(XB, B_PER_W - s * XB)
        if nb <= 0:
            return
        xfill_cp[s % 2] = pltpu.async_copy(
            x_hbm.at[pl.ds((base + s * XB) * L, nb * L)],
            xqs[s % 2].at[pl.ds(0, nb * L)], xsem)

    def start_gather(j):
        s = j // XB
        if xfill_cp[s % 2] is not None:
            xfill_cp[s % 2].wait()
            xfill_cp[s % 2] = None
        k = j % NSL
        gather_cp[k] = pltpu.async_copy(
            table_hbm.at[xqs[s % 2].at[pl.ds((j % XB) * L, L)]],
            bufs[k], gsems[k])

    def add_body_for(buf):
        def add_body(t, _):
            for h in range(NVH):
                sl = pl.ds(h * 16, 16)
                buf[t, sl] = buf[t, sl] + pos_v[t, sl]
            return _
        return add_body

    pltpu.sync_copy(x_hbm.at[pl.ds(base * L, XB * L)], xq0)
    start_gather(0)
    start_gather(1)
    start_xfill(1)
    pltpu.sync_copy(pos_hbm.at[pl.ds(0, L)], pos_v)

    def scale_body(t, _):
        for h in range(NVH):
            sl = pl.ds(h * 16, 16)
            pos_v[t, sl] = pos_v[t, sl] * SCALE
        return _

    lax.fori_loop(0, L, scale_body, 0)

    for j in range(B_PER_W):
        k = j % NSL
        buf, osem = bufs[k], osems[k]
        gather_cp[k].wait()
        lax.fori_loop(0, L, add_body_for(buf), 0)
        out_cp[k] = pltpu.async_copy(
            buf, out_hbm.at[pl.ds((base + j) * L, L)], osem)
        if (j + 1) % XB == 0:
            start_xfill(j // XB + 2)
        if j + 2 < B_PER_W:
            nk = (j + 2) % NSL
            if out_cp[nk] is not None:
                out_cp[nk].wait()
            start_gather(j + 2)

    for k in range(NSL):
        if out_cp[k] is not None:
            out_cp[k].wait()


@jax.jit
def _emb(x_flat, table, pos_weight):
    mesh = plsc.VectorSubcoreMesh(core_axis_name="c", subcore_axis_name="s",
                                  num_cores=NC, num_subcores=NS)
    return pl.kernel(
        _emb_body,
        out_type=jax.ShapeDtypeStruct((B * L, HIDDEN), jnp.float32),
        mesh=mesh,
        scratch_types=[
            pltpu.VMEM((L, HIDDEN), jnp.float32),
            pltpu.VMEM((L, HIDDEN), jnp.float32),
            pltpu.VMEM((L, HIDDEN), jnp.float32),
            pltpu.VMEM((L, HIDDEN), jnp.float32),
            pltpu.VMEM((L, HIDDEN), jnp.float32),
            pltpu.VMEM((XB * L,), jnp.int32),
            pltpu.VMEM((XB * L,), jnp.int32),
            pltpu.SemaphoreType.DMA,
            pltpu.SemaphoreType.DMA,
            pltpu.SemaphoreType.DMA,
            pltpu.SemaphoreType.DMA,
            pltpu.SemaphoreType.DMA,
            pltpu.SemaphoreType.DMA,
            pltpu.SemaphoreType.DMA,
            pltpu.SemaphoreType.DMA,
            pltpu.SemaphoreType.DMA,
        ],
    )(x_flat, table, pos_weight)


def kernel(X, table, pos_weight):
    x_flat = X.reshape(B * L).astype(jnp.int32)
    out = _emb(x_flat, table, pos_weight)
    return out.reshape(B, L, HIDDEN)
